# baseline (speedup 1.0000x reference)
import functools
import math

import jax
import jax.numpy as jnp
from jax import lax
from jax.experimental import pallas as pl
from jax.experimental.pallas import tpu as pltpu
from jax.experimental.pallas import tpu_sc as plsc

_R = 16
_N = 32768
_K = 8
_M = 9
_EPS = 1e-2
_INV_EPS = 100.0
_ITERS = 10
_HALF = _N // 2
_VSTEPS = _HALF // 16
_LN2 = math.log(2.0)

_Y = [10.0 * j / (_M - 1) for j in range(_M)]
_LB = [math.log((_N - _K) / _N)] + [math.log(1.0 / _N)] * _K
_LA = -math.log(_N)
_NEG_BIG = -3.0e38


def _gather(x, idx):
    return x.at[idx].get(mode="promise_in_bounds")


def _lane_sum(x):
    for d in (1, 2, 4, 8):
        x = x + _gather(x, lax.iota(jnp.int32, 16) ^ d)
    return x


def _lane_max(x):
    for d in (1, 2, 4, 8):
        x = jnp.maximum(x, _gather(x, lax.iota(jnp.int32, 16) ^ d))
    return x


def _bcast(x, j):
    return _gather(x, jnp.full((16,), j, jnp.int32))


def _lane(vec, j, val):
    return jnp.where(lax.iota(jnp.int32, 16) == j, val, vec)


def _vlog(s):
    bits = lax.bitcast_convert_type(s, jnp.int32).astype(jnp.float32)
    y = bits * (_LN2 / 2.0 ** 23) - (126.9569 * _LN2)
    y = y + s * jnp.exp(-y) - 1.0
    y = y + s * jnp.exp(-y) - 1.0
    return y


def _vsqrt(v):
    bits = lax.bitcast_convert_type(v, jnp.int32)
    s = lax.bitcast_convert_type(
        lax.shift_right_arithmetic(bits, 1) + jnp.int32(0x1FBD1DF5), jnp.float32)
    for _ in range(3):
        s = 0.5 * (s + v / s)
    return s


def _soft_topk_body(z_hbm, out_hbm, z_ref, zq_ref, m_ref, is_ref,
                    exw_ref, exr_ref, ob_ref, shared_ref):
    c = lax.axis_index("c")
    s = lax.axis_index("s")
    row = c * (_R // 2) + (s // 2)
    half = s % 2
    base = row * _N + half * _HALF

    pltpu.sync_copy(z_hbm.at[pl.ds(base, _HALF)], z_ref)

    def exchange(vec, combine):
        exw_ref[...] = vec
        pltpu.sync_copy(exw_ref, shared_ref.at[16 + s])
        plsc.subcore_barrier()
        pltpu.sync_copy(shared_ref.at[pl.ds(16, 16)], exr_ref)
        plsc.subcore_barrier()
        return combine(vec, exr_ref[s ^ 1, :])

    def chunk(ref, i):
        return ref[pl.ds(pl.multiple_of(i * 16, 16), 16)]

    def p1(i, acc):
        return acc + chunk(z_ref, i)
    acc = lax.fori_loop(0, _VSTEPS, p1, jnp.zeros((16,), jnp.float32))
    mu = exchange(_lane_sum(acc), lambda a, b: a + b) * (1.0 / _N)

    def p2(i, acc):
        d = chunk(z_ref, i) - mu
        return acc + d * d
    acc = lax.fori_loop(0, _VSTEPS, p2, jnp.zeros((16,), jnp.float32))
    var_v = exchange(_lane_sum(acc), lambda a, b: a + b) * (1.0 / _N)
    inv_sigma = 1.0 / (_vsqrt(var_v) + 1e-12)

    def p3(i, qs):
        zv = chunk(z_ref, i)
        x = (zv - mu) * inv_sigma
        zq = 10.0 / (1.0 + jnp.exp(-x))
        idx = pl.ds(pl.multiple_of(i * 16, 16), 16)
        zq_ref[idx] = zq
        m_ref[idx] = jnp.zeros((16,), jnp.float32)
        is_ref[idx] = jnp.ones((16,), jnp.float32)
        out = []
        for j in range(_M):
            d = zq - _Y[j]
            out.append(jnp.maximum(qs[j], -(d * d)))
        return tuple(out)

    qs = lax.fori_loop(0, _VSTEPS, p3,
                       tuple(jnp.full((16,), _NEG_BIG) for _ in range(_M)))
    qv = jnp.full((16,), _NEG_BIG)
    for j in range(_M):
        qv = _lane(qv, j, _lane_max(qs[j]))
    n_v = exchange(qv, jnp.maximum)

    lb_v = jnp.zeros((16,), jnp.float32)
    for j in range(_M):
        lb_v = _lane(lb_v, j, _LB[j])

    def sink(_, carry):
        g_v, n_v = carry

        negn = [-_bcast(n_v, j) for j in range(_M)]

        def gstep(i, accs):
            zq = chunk(zq_ref, i)
            mv = chunk(m_ref, i)
            iv = chunk(is_ref, i)
            out = []
            for j in range(_M):
                d = zq - _Y[j]
                w = (negn[j] - mv) - d * d
                out.append(accs[j] + jnp.exp(w) * iv)
            return tuple(out)

        accs = lax.fori_loop(0, _VSTEPS, gstep,
                             tuple(jnp.zeros((16,), jnp.float32)
                                   for _ in range(_M)))
        sv = jnp.ones((16,), jnp.float32)
        for j in range(_M):
            sv = _lane(sv, j, _lane_sum(accs[j]))
        sv = exchange(sv, lambda a, b: a + b)
        g_v = -_EPS * (_vlog(sv) + n_v + _LA)

        cj = [_bcast(g_v, j) * _INV_EPS + _LB[j] for j in range(_M)]

        def fstep(i, qs):
            zq = chunk(zq_ref, i)
            ts = []
            for j in range(_M):
                d = zq - _Y[j]
                ts.append(cj[j] - d * d)
            mv = ts[0]
            for j in range(1, _M):
                mv = jnp.maximum(mv, ts[j])
            sig = jnp.zeros((16,), jnp.float32)
            out = []
            for j in range(_M):
                dv = ts[j] - mv
                sig = sig + jnp.exp(dv)
                out.append(jnp.maximum(qs[j], dv))
            idx = pl.ds(pl.multiple_of(i * 16, 16), 16)
            m_ref[idx] = mv
            is_ref[idx] = 1.0 / sig
            return tuple(out)

        qs = lax.fori_loop(0, _VSTEPS, fstep,
                           tuple(jnp.full((16,), _NEG_BIG)
                                 for _ in range(_M)))
        qv = jnp.full((16,), _NEG_BIG)
        for j in range(_M):
            qv = _lane(qv, j, _lane_max(qs[j]))
        qv = exchange(qv, jnp.maximum)
        n_v = qv - g_v * _INV_EPS - lb_v
        return g_v, n_v

    g_v, n_v = lax.fori_loop(0, _ITERS, sink,
                             (jnp.zeros((16,), jnp.float32), n_v))

    negn = [-_bcast(n_v, j) for j in range(1, _M)]

    def estep(i, accs):
        zq = chunk(zq_ref, i)
        mv = chunk(m_ref, i)
        iv = chunk(is_ref, i)
        zv = chunk(z_ref, i)
        wiz = iv * zv
        out = []
        for j in range(1, _M):
            d = zq - _Y[j]
            w = (negn[j - 1] - mv) - d * d
            out.append(accs[j - 1] + jnp.exp(w) * wiz)
        return tuple(out)

    accs = lax.fori_loop(0, _VSTEPS, estep,
                         tuple(jnp.zeros((16,), jnp.float32)
                               for _ in range(_K)))
    pv = jnp.zeros((16,), jnp.float32)
    for j in range(1, _M):
        pv = _lane(pv, j - 1, _lane_sum(accs[j - 1]))
    pv = exchange(pv, lambda a, b: a + b)
    lv = jnp.zeros((16,), jnp.float32)
    for j in range(1, _M):
        lv = _lane(lv, j - 1,
                   _bcast(g_v, j) * _INV_EPS + _LA + _bcast(n_v, j))
    ob_ref[...] = jnp.exp(lv) * pv

    @pl.when(half == 0)
    def _():
        pltpu.sync_copy(ob_ref, out_hbm.at[row])


_soft_topk = functools.partial(
    pl.kernel,
    mesh=plsc.VectorSubcoreMesh(core_axis_name="c", subcore_axis_name="s"),
    out_type=jax.ShapeDtypeStruct((_R, 16), jnp.float32),
    scratch_types=[
        pltpu.VMEM((_HALF,), jnp.float32),
        pltpu.VMEM((_HALF,), jnp.float32),
        pltpu.VMEM((_HALF,), jnp.float32),
        pltpu.VMEM((_HALF,), jnp.float32),
        pltpu.VMEM((16,), jnp.float32),
        pltpu.VMEM((16, 16), jnp.float32),
        pltpu.VMEM((16,), jnp.float32),
        pltpu.VMEM_SHARED((32, 16), jnp.float32),
    ],
)(_soft_topk_body)


def kernel(scores):
    out = _soft_topk(scores.reshape(-1))
    return out[:, :_K]
